# baseline (speedup 1.0000x reference)
import jax
import jax.numpy as jnp
from jax.experimental import pallas as pl
from jax.experimental.pallas import tpu as pltpu
import functools

_K = 1024
_D = 64
_BETA = 0.25
_BT = 512


def _vq_kernel(x_ref, w_ref, loss_ref, q_ref, perp_ref, enc_ref,
               hist_acc, loss_acc, *, n_tokens, n_blocks):
    i = pl.program_id(0)
    xb = x_ref[...]
    w = w_ref[...]

    x2 = jnp.sum(xb * xb, axis=1, keepdims=True)
    w2 = jnp.sum(w * w, axis=1, keepdims=True).T
    xw = jax.lax.dot_general(xb, w, (((1,), (1,)), ((), ())),
                             preferred_element_type=jnp.float32)
    dist = x2 - 2.0 * xw + w2
    dmin = jnp.min(dist, axis=1, keepdims=True)
    cols = jax.lax.broadcasted_iota(jnp.int32, (_BT, _K), 1)
    idx = jnp.min(jnp.where(dist == dmin, cols, _K), axis=1)

    enc = (cols == idx[:, None]).astype(jnp.float32)
    enc_ref[...] = enc

    q = jax.lax.dot_general(enc, w, (((1,), (0,)), ((), ())),
                            preferred_element_type=jnp.float32)
    q_ref[...] = q

    diff = xb - q
    block_loss = jnp.sum(diff * diff).reshape(1, 1)
    block_hist = jnp.sum(enc, axis=0, keepdims=True)

    @pl.when(i == 0)
    def _init():
        hist_acc[...] = block_hist
        loss_acc[...] = block_loss

    @pl.when(i != 0)
    def _accum():
        hist_acc[...] += block_hist
        loss_acc[...] += block_loss

    @pl.when(i == n_blocks - 1)
    def _finalize():
        loss_ref[...] = loss_acc[...] * ((1.0 + _BETA) / (n_tokens * _D))
        avg = hist_acc[...] / n_tokens
        perp_ref[...] = jnp.exp(-jnp.sum(avg * jnp.log(avg + 1e-10))).reshape(1, 1)


def kernel(x, W):
    flat = x.reshape(-1, _D)
    n_tokens = flat.shape[0]
    n_blocks = n_tokens // _BT

    grid = (n_blocks,)
    kfn = functools.partial(_vq_kernel, n_tokens=n_tokens, n_blocks=n_blocks)
    loss, quantized, perp, enc = pl.pallas_call(
        kfn,
        grid=grid,
        in_specs=[
            pl.BlockSpec((_BT, _D), lambda i: (i, 0)),
            pl.BlockSpec((_K, _D), lambda i: (0, 0)),
        ],
        out_specs=[
            pl.BlockSpec((1, 1), lambda i: (0, 0)),
            pl.BlockSpec((_BT, _D), lambda i: (i, 0)),
            pl.BlockSpec((1, 1), lambda i: (0, 0)),
            pl.BlockSpec((_BT, _K), lambda i: (i, 0)),
        ],
        out_shape=[
            jax.ShapeDtypeStruct((1, 1), jnp.float32),
            jax.ShapeDtypeStruct((n_tokens, _D), jnp.float32),
            jax.ShapeDtypeStruct((1, 1), jnp.float32),
            jax.ShapeDtypeStruct((n_tokens, _K), jnp.float32),
        ],
        scratch_shapes=[
            pltpu.VMEM((1, _K), jnp.float32),
            pltpu.VMEM((1, 1), jnp.float32),
        ],
        compiler_params=pltpu.CompilerParams(
            dimension_semantics=("arbitrary",),
        ),
    )(flat, W)

    return (loss[0, 0], quantized, perp[0, 0], enc)


# speedup vs baseline: 1.0401x; 1.0401x over previous
import jax
import jax.numpy as jnp
from jax.experimental import pallas as pl
from jax.experimental.pallas import tpu as pltpu
import functools

_K = 1024
_D = 64
_BETA = 0.25
_BT = 512


def _vq_kernel(x_ref, w_ref, loss_ref, q_ref, perp_ref, enc_ref,
               hist_acc, loss_acc, w2_acc, *, n_tokens, n_blocks):
    i = pl.program_id(0)
    xb = x_ref[...]
    w = w_ref[...]

    @pl.when(i == 0)
    def _w2_once():
        w2_acc[...] = jnp.sum(w * w, axis=1, keepdims=True).T

    x2 = jnp.sum(xb * xb, axis=1, keepdims=True)
    w2 = w2_acc[...]
    xw = jax.lax.dot_general(xb, w, (((1,), (1,)), ((), ())),
                             preferred_element_type=jnp.float32)
    dist = x2 - 2.0 * xw + w2
    dmin = jnp.min(dist, axis=1, keepdims=True)
    cols = jax.lax.broadcasted_iota(jnp.int32, (_BT, _K), 1)
    idx = jnp.min(jnp.where(dist == dmin, cols, _K), axis=1)

    enc = (cols == idx[:, None]).astype(jnp.float32)
    enc_ref[...] = enc

    q = jax.lax.dot_general(enc, w, (((1,), (0,)), ((), ())),
                            preferred_element_type=jnp.float32)
    q_ref[...] = q

    block_loss = jnp.sum(dmin).reshape(1, 1)
    block_hist = jnp.sum(enc, axis=0, keepdims=True)

    @pl.when(i == 0)
    def _init():
        hist_acc[...] = block_hist
        loss_acc[...] = block_loss

    @pl.when(i != 0)
    def _accum():
        hist_acc[...] += block_hist
        loss_acc[...] += block_loss

    @pl.when(i == n_blocks - 1)
    def _finalize():
        loss_ref[...] = loss_acc[...] * ((1.0 + _BETA) / (n_tokens * _D))
        avg = hist_acc[...] / n_tokens
        perp_ref[...] = jnp.exp(-jnp.sum(avg * jnp.log(avg + 1e-10))).reshape(1, 1)


def kernel(x, W):
    flat = x.reshape(-1, _D)
    n_tokens = flat.shape[0]
    n_blocks = n_tokens // _BT

    grid = (n_blocks,)
    kfn = functools.partial(_vq_kernel, n_tokens=n_tokens, n_blocks=n_blocks)
    loss, quantized, perp, enc = pl.pallas_call(
        kfn,
        grid=grid,
        in_specs=[
            pl.BlockSpec((_BT, _D), lambda i: (i, 0)),
            pl.BlockSpec((_K, _D), lambda i: (0, 0)),
        ],
        out_specs=[
            pl.BlockSpec((1, 1), lambda i: (0, 0)),
            pl.BlockSpec((_BT, _D), lambda i: (i, 0)),
            pl.BlockSpec((1, 1), lambda i: (0, 0)),
            pl.BlockSpec((_BT, _K), lambda i: (i, 0)),
        ],
        out_shape=[
            jax.ShapeDtypeStruct((1, 1), jnp.float32),
            jax.ShapeDtypeStruct((n_tokens, _D), jnp.float32),
            jax.ShapeDtypeStruct((1, 1), jnp.float32),
            jax.ShapeDtypeStruct((n_tokens, _K), jnp.float32),
        ],
        scratch_shapes=[
            pltpu.VMEM((1, _K), jnp.float32),
            pltpu.VMEM((1, 1), jnp.float32),
            pltpu.VMEM((1, _K), jnp.float32),
        ],
        compiler_params=pltpu.CompilerParams(
            dimension_semantics=("arbitrary",),
        ),
    )(flat, W)

    return (loss[0, 0], quantized, perp[0, 0], enc)
